# baseline (speedup 1.0000x reference)
import functools

import jax
import jax.numpy as jnp
import numpy as np
from jax import lax
from jax.experimental import pallas as pl
from jax.experimental.pallas import tpu as pltpu
from jax.experimental.pallas import tpu_sc as plsc

_N, _H, _W = 8, 32, 32
_L = _H * _W
_CH, _G = 192, 12
_GC = _CH // _G
_P = 9
_OMD = int(np.ceil(_G * _P * 3 / 8) * 8)
_ROWS_A = _CH + _G * 32

_perm = np.zeros((_G * 32,), np.int32)
_keep = np.zeros((_G * 32, 1), np.float32)
_bshift = np.zeros((_G * 32,), np.float32)
for _g in range(_G):
    for _r in range(27):
        if _r < 9:
            _m = 2 * _r
            _bshift[_g * 32 + _r] = _r % 3 - 1
        elif _r < 18:
            _m = 2 * (_r - 9) + 1
            _bshift[_g * 32 + _r] = (_r - 9) // 3 - 1
        else:
            _m = _r
        _perm[_g * 32 + _r] = _g * 27 + _m
        _keep[_g * 32 + _r, 0] = 1.0


def _proj_body(w_ref, x_ref, b_ref, o_ref):
    o_ref[0] = (
        jnp.dot(w_ref[...], x_ref[0], preferred_element_type=jnp.float32)
        + b_ref[...]
    )


def _proj(w, x, b, rows):
    return pl.pallas_call(
        _proj_body,
        grid=(_N,),
        in_specs=[
            pl.BlockSpec((rows, _CH), lambda n: (0, 0)),
            pl.BlockSpec((1, _CH, _L), lambda n: (n, 0, 0)),
            pl.BlockSpec((rows, 1), lambda n: (0, 0)),
        ],
        out_specs=pl.BlockSpec((1, rows, _L), lambda n: (n, 0, 0)),
        out_shape=jax.ShapeDtypeStruct((_N, rows, _L), jnp.float32),
    )(w, x, b)


_mesh = plsc.VectorSubcoreMesh(core_axis_name="c", subcore_axis_name="s")


@functools.partial(
    pl.kernel,
    mesh=_mesh,
    out_type=jax.ShapeDtypeStruct((_N, _CH, _L), jnp.float32),
    scratch_types=[
        pltpu.VMEM((_GC, _L), jnp.float32),
        pltpu.VMEM((32, _L), jnp.float32),
        pltpu.VMEM((_GC, _L), jnp.float32),
    ],
    compiler_params=pltpu.CompilerParams(
        use_tc_tiling_on_sc=False, needs_layout_passes=False
    ),
)
def _sc_sample(a_hbm, out_hbm, xvt, comp, outv):
    wid = lax.axis_index("s") * 2 + lax.axis_index("c")
    n = wid >> 2
    j = wid & 3

    def tbody(t, carry):
        g = j * 3 + t
        pltpu.sync_copy(a_hbm.at[n, pl.ds(g * _GC, _GC), :], xvt)
        pltpu.sync_copy(a_hbm.at[n, pl.ds(_CH + g * 32, 32), :], comp)

        def bbody(b, c2):
            for sub in range(2):
                l0 = b * 32 + sub * 16
                li = lax.broadcasted_iota(jnp.int32, (16,), 0) + l0
                pix_y = (li >> 5).astype(jnp.float32)
                pix_x = (li & 31).astype(jnp.float32)
                acc = [jnp.zeros((16,), jnp.float32) for _ in range(_GC)]
                for p in range(_P):
                    locx = pix_x + comp[p, pl.ds(l0, 16)]
                    locy = pix_y + comp[9 + p, pl.ds(l0, 16)]
                    msk = comp[18 + p, pl.ds(l0, 16)]
                    xt = locx.astype(jnp.int32).astype(jnp.float32)
                    yt = locy.astype(jnp.int32).astype(jnp.float32)
                    x0 = jnp.where(locx < xt, xt - 1.0, xt)
                    y0 = jnp.where(locy < yt, yt - 1.0, yt)
                    lx = locx - x0
                    ly = locy - y0
                    hx = 1.0 - lx
                    hym = (1.0 - ly) * msk
                    lym = ly * msk
                    x1 = x0 + 1.0
                    y1 = y0 + 1.0
                    vx0 = (x0 >= 0.0) & (x0 < float(_W))
                    vx1 = (x1 >= 0.0) & (x1 < float(_W))
                    vy0 = (y0 >= 0.0) & (y0 < float(_H))
                    vy1 = (y1 >= 0.0) & (y1 < float(_H))
                    xc0 = jnp.clip(x0, 0.0, float(_W - 1))
                    xc1 = jnp.clip(x1, 0.0, float(_W - 1))
                    ly0 = jnp.clip(y0, 0.0, float(_H - 1)) * float(_W)
                    ly1 = jnp.clip(y1, 0.0, float(_H - 1)) * float(_W)
                    for liny, vy, xc, vx, bw in (
                        (ly0, vy0, xc0, vx0, hym * hx),
                        (ly0, vy0, xc1, vx1, hym * lx),
                        (ly1, vy1, xc0, vx0, lym * hx),
                        (ly1, vy1, xc1, vx1, lym * lx),
                    ):
                        lin = (liny + xc).astype(jnp.int32)
                        wv = jnp.where(vy & vx, bw, 0.0)
                        for ch in range(_GC):
                            cvec = jnp.full((16,), ch, jnp.int32)
                            val = plsc.load_gather(xvt, [cvec, lin])
                            acc[ch] = acc[ch] + wv * val
                for ch in range(_GC):
                    outv[ch, pl.ds(l0, 16)] = acc[ch]
            return c2

        lax.fori_loop(0, _L // 32, bbody, 0)
        pltpu.sync_copy(outv, out_hbm.at[n, pl.ds(g * _GC, _GC), :])
        return carry

    lax.fori_loop(0, 3, tbody, 0)


def kernel(input, value_w, value_b, offset_mask_w, offset_mask_b, output_w, output_b):
    x_t = jnp.transpose(input, (0, 2, 1))
    w2 = offset_mask_w[_perm] * _keep
    b2 = offset_mask_b[_perm] * _keep[:, 0] + _bshift
    w_all = jnp.concatenate([value_w, w2], axis=0)
    b_all = jnp.concatenate([value_b, b2], axis=0)[:, None]
    a = _proj(w_all, x_t, b_all, _ROWS_A)
    s = _sc_sample(a)
    c = _proj(output_w, s, output_b[:, None], _CH)
    return jnp.transpose(c, (0, 2, 1))


# speedup vs baseline: 1.2891x; 1.2891x over previous
import functools

import jax
import jax.numpy as jnp
import numpy as np
from jax import lax
from jax.experimental import pallas as pl
from jax.experimental.pallas import tpu as pltpu
from jax.experimental.pallas import tpu_sc as plsc

_N, _H, _W = 8, 32, 32
_L = _H * _W
_CH, _G = 192, 12
_GC = _CH // _G
_P = 9
_OMD = int(np.ceil(_G * _P * 3 / 8) * 8)
_ROWS_A = _CH + _G * 32

_perm = np.zeros((_G * 32,), np.int32)
_keep = np.zeros((_G * 32, 1), np.float32)
_bshift = np.zeros((_G * 32,), np.float32)
for _g in range(_G):
    for _r in range(27):
        if _r < 9:
            _m = 2 * _r
            _bshift[_g * 32 + _r] = _r % 3 - 1
        elif _r < 18:
            _m = 2 * (_r - 9) + 1
            _bshift[_g * 32 + _r] = (_r - 9) // 3 - 1
        else:
            _m = _r
        _perm[_g * 32 + _r] = _g * 27 + _m
        _keep[_g * 32 + _r, 0] = 1.0


def _proj_body(w_ref, x_ref, b_ref, o_ref):
    o_ref[0] = (
        jnp.dot(w_ref[...], x_ref[0], preferred_element_type=jnp.float32)
        + b_ref[...]
    )


def _proj(w, x, b, rows):
    return pl.pallas_call(
        _proj_body,
        grid=(_N,),
        in_specs=[
            pl.BlockSpec((rows, _CH), lambda n: (0, 0)),
            pl.BlockSpec((1, _CH, _L), lambda n: (n, 0, 0)),
            pl.BlockSpec((rows, 1), lambda n: (0, 0)),
        ],
        out_specs=pl.BlockSpec((1, rows, _L), lambda n: (n, 0, 0)),
        out_shape=jax.ShapeDtypeStruct((_N, rows, _L), jnp.float32),
    )(w, x, b)


def _proj_px_body(x_ref, wt_ref, b_ref, o_ref):
    o_ref[0] = (
        jnp.dot(x_ref[0], wt_ref[...], preferred_element_type=jnp.float32)
        + b_ref[...]
    )


def _proj_px(x, wt, b):
    return pl.pallas_call(
        _proj_px_body,
        grid=(_N,),
        in_specs=[
            pl.BlockSpec((1, _L, _CH), lambda n: (n, 0, 0)),
            pl.BlockSpec((_CH, _CH), lambda n: (0, 0)),
            pl.BlockSpec((1, _CH), lambda n: (0, 0)),
        ],
        out_specs=pl.BlockSpec((1, _L, _CH), lambda n: (n, 0, 0)),
        out_shape=jax.ShapeDtypeStruct((_N, _L, _CH), jnp.float32),
    )(x, wt, b)


_mesh = plsc.VectorSubcoreMesh(core_axis_name="c", subcore_axis_name="s")


@functools.partial(
    pl.kernel,
    mesh=_mesh,
    out_type=jax.ShapeDtypeStruct((_N, _L, _CH), jnp.float32),
    scratch_types=[
        pltpu.VMEM((_GC, _L), jnp.float32),
        pltpu.VMEM((32, _L), jnp.float32),
        pltpu.VMEM((_L, _GC), jnp.float32),
    ],
    compiler_params=pltpu.CompilerParams(
        use_tc_tiling_on_sc=False, needs_layout_passes=False
    ),
)
def _sc_sample(a_hbm, out_hbm, xvt, comp, outv):
    wid = lax.axis_index("s") * 2 + lax.axis_index("c")
    n = wid >> 2
    j = wid & 3

    def tbody(t, carry):
        g = j * 3 + t
        pltpu.sync_copy(a_hbm.at[n, pl.ds(g * _GC, _GC), :], xvt)
        pltpu.sync_copy(a_hbm.at[n, pl.ds(_CH + g * 32, 32), :], comp)

        def bbody(b, c2):
            for sub in range(1):
                l0 = b * 16
                li = lax.broadcasted_iota(jnp.int32, (16,), 0) + l0
                pix_y = (li >> 5).astype(jnp.float32)
                pix_x = (li & 31).astype(jnp.float32)
                acc = [jnp.zeros((16,), jnp.float32) for _ in range(_GC)]
                for p in range(_P):
                    locx = pix_x + comp[p, pl.ds(l0, 16)]
                    locy = pix_y + comp[9 + p, pl.ds(l0, 16)]
                    msk = comp[18 + p, pl.ds(l0, 16)]
                    xt = locx.astype(jnp.int32).astype(jnp.float32)
                    yt = locy.astype(jnp.int32).astype(jnp.float32)
                    x0 = jnp.where(locx < xt, xt - 1.0, xt)
                    y0 = jnp.where(locy < yt, yt - 1.0, yt)
                    lx = locx - x0
                    ly = locy - y0
                    hx = 1.0 - lx
                    hym = (1.0 - ly) * msk
                    lym = ly * msk
                    x1 = x0 + 1.0
                    y1 = y0 + 1.0
                    vx0 = (x0 >= 0.0) & (x0 < float(_W))
                    vx1 = (x1 >= 0.0) & (x1 < float(_W))
                    vy0 = (y0 >= 0.0) & (y0 < float(_H))
                    vy1 = (y1 >= 0.0) & (y1 < float(_H))
                    xc0 = jnp.clip(x0, 0.0, float(_W - 1))
                    xc1 = jnp.clip(x1, 0.0, float(_W - 1))
                    ly0 = jnp.clip(y0, 0.0, float(_H - 1)) * float(_W)
                    ly1 = jnp.clip(y1, 0.0, float(_H - 1)) * float(_W)
                    for liny, vy, xc, vx, bw in (
                        (ly0, vy0, xc0, vx0, hym * hx),
                        (ly0, vy0, xc1, vx1, hym * lx),
                        (ly1, vy1, xc0, vx0, lym * hx),
                        (ly1, vy1, xc1, vx1, lym * lx),
                    ):
                        lin = (liny + xc).astype(jnp.int32)
                        wv = jnp.where(vy & vx, bw, 0.0)
                        for ch in range(_GC):
                            cvec = jnp.full((16,), ch, jnp.int32)
                            val = plsc.load_gather(xvt, [cvec, lin])
                            acc[ch] = acc[ch] + wv * val
                for ch in range(_GC):
                    cvec = jnp.full((16,), ch, jnp.int32)
                    plsc.store_scatter(outv, [li, cvec], acc[ch])
            return c2

        lax.fori_loop(0, _L // 16, bbody, 0)
        pltpu.sync_copy(outv, out_hbm.at[n, :, pl.ds(g * _GC, _GC)])
        return carry

    lax.fori_loop(0, 3, tbody, 0)


def kernel(input, value_w, value_b, offset_mask_w, offset_mask_b, output_w, output_b):
    x_t = jnp.transpose(input, (0, 2, 1))
    w2 = offset_mask_w[_perm] * _keep
    b2 = offset_mask_b[_perm] * _keep[:, 0] + _bshift
    w_all = jnp.concatenate([value_w, w2], axis=0)
    b_all = jnp.concatenate([value_b, b2], axis=0)[:, None]
    a = _proj(w_all, x_t, b_all, _ROWS_A)
    s = _sc_sample(a)
    return _proj_px(s, jnp.transpose(output_w), output_b[None, :])


# speedup vs baseline: 1.4075x; 1.0919x over previous
import functools

import jax
import jax.numpy as jnp
import numpy as np
from jax import lax
from jax.experimental import pallas as pl
from jax.experimental.pallas import tpu as pltpu
from jax.experimental.pallas import tpu_sc as plsc

_N, _H, _W = 8, 32, 32
_L = _H * _W
_CH, _G = 192, 12
_GC = _CH // _G
_P = 9
_OMD = int(np.ceil(_G * _P * 3 / 8) * 8)
_ROWS_A = _CH + _G * 32

_perm = np.zeros((_G * 32,), np.int32)
_keep = np.zeros((_G * 32, 1), np.float32)
_bshift = np.zeros((_G * 32,), np.float32)
for _g in range(_G):
    for _r in range(27):
        if _r < 9:
            _m = 2 * _r
            _bshift[_g * 32 + _r] = _r % 3 - 1
        elif _r < 18:
            _m = 2 * (_r - 9) + 1
            _bshift[_g * 32 + _r] = (_r - 9) // 3 - 1
        else:
            _m = _r
        _perm[_g * 32 + _r] = _g * 27 + _m
        _keep[_g * 32 + _r, 0] = 1.0


def _proj_body(w_ref, x_ref, b_ref, o_ref):
    o_ref[0] = (
        jnp.dot(w_ref[...], x_ref[0], preferred_element_type=jnp.float32)
        + b_ref[...]
    )


def _proj(w, x, b, rows):
    return pl.pallas_call(
        _proj_body,
        grid=(_N,),
        in_specs=[
            pl.BlockSpec((rows, _CH), lambda n: (0, 0)),
            pl.BlockSpec((1, _CH, _L), lambda n: (n, 0, 0)),
            pl.BlockSpec((rows, 1), lambda n: (0, 0)),
        ],
        out_specs=pl.BlockSpec((1, rows, _L), lambda n: (n, 0, 0)),
        out_shape=jax.ShapeDtypeStruct((_N, rows, _L), jnp.float32),
    )(w, x, b)


def _proj_px_body(x_ref, wt_ref, b_ref, o_ref):
    o_ref[0] = (
        jnp.dot(x_ref[0], wt_ref[...], preferred_element_type=jnp.float32)
        + b_ref[...]
    )


def _proj_px(x, wt, b):
    return pl.pallas_call(
        _proj_px_body,
        grid=(_N,),
        in_specs=[
            pl.BlockSpec((1, _L, _CH), lambda n: (n, 0, 0)),
            pl.BlockSpec((_CH, _CH), lambda n: (0, 0)),
            pl.BlockSpec((1, _CH), lambda n: (0, 0)),
        ],
        out_specs=pl.BlockSpec((1, _L, _CH), lambda n: (n, 0, 0)),
        out_shape=jax.ShapeDtypeStruct((_N, _L, _CH), jnp.float32),
    )(x, wt, b)


_mesh = plsc.VectorSubcoreMesh(core_axis_name="c", subcore_axis_name="s")


@functools.partial(
    pl.kernel,
    mesh=_mesh,
    out_type=jax.ShapeDtypeStruct((_N, _CH, _L), jnp.float32),
    scratch_types=[
        pltpu.VMEM((_GC, _L), jnp.float32),
        pltpu.VMEM((_GC, _L), jnp.float32),
        pltpu.VMEM((27, _L), jnp.float32),
        pltpu.VMEM((27, _L), jnp.float32),
        pltpu.VMEM((_GC, _L), jnp.float32),
        pltpu.VMEM((_GC, _L), jnp.float32),
        pltpu.SemaphoreType.DMA,
        pltpu.SemaphoreType.DMA,
        pltpu.SemaphoreType.DMA,
        pltpu.SemaphoreType.DMA,
    ],
    compiler_params=pltpu.CompilerParams(
        use_tc_tiling_on_sc=False, needs_layout_passes=False
    ),
)
def _sc_sample(
    a_hbm, out_hbm, xvt0, xvt1, comp0, comp1, outv0, outv1, si0, si1, so0, so1
):
    wid = lax.axis_index("s") * 2 + lax.axis_index("c")
    n = wid >> 2
    j = wid & 3

    xv = (xvt0, xvt1)
    cp = (comp0, comp1)
    ov = (outv0, outv1)
    si = (si0, si1)
    so = (so0, so1)

    def start_in(t, slot):
        g = j * 3 + t
        return (
            pltpu.async_copy(a_hbm.at[n, pl.ds(g * _GC, _GC), :], xv[slot], si[slot]),
            pltpu.async_copy(
                a_hbm.at[n, pl.ds(_CH + g * 32, 27), :], cp[slot], si[slot]
            ),
        )

    pending_in = start_in(0, 0)
    pending_out = [None, None]
    for t in range(3):
        slot = t % 2
        xvt = xv[slot]
        comp = cp[slot]
        outv = ov[slot]
        for h in pending_in:
            h.wait()
        if t < 2:
            pending_in = start_in(t + 1, 1 - slot)
        if pending_out[slot] is not None:
            pending_out[slot].wait()

        def bbody(b, c2, xvt=xvt, comp=comp, outv=outv):
            for sub in range(1):
                l0 = b * 16
                li = lax.broadcasted_iota(jnp.int32, (16,), 0) + l0
                pix_y = (li >> 5).astype(jnp.float32)
                pix_x = (li & 31).astype(jnp.float32)
                acc = [jnp.zeros((16,), jnp.float32) for _ in range(_GC)]
                for p in range(_P):
                    locx = pix_x + comp[p, pl.ds(l0, 16)]
                    locy = pix_y + comp[9 + p, pl.ds(l0, 16)]
                    msk = comp[18 + p, pl.ds(l0, 16)]
                    xt = locx.astype(jnp.int32).astype(jnp.float32)
                    yt = locy.astype(jnp.int32).astype(jnp.float32)
                    x0 = jnp.where(locx < xt, xt - 1.0, xt)
                    y0 = jnp.where(locy < yt, yt - 1.0, yt)
                    lx = locx - x0
                    ly = locy - y0
                    hx = 1.0 - lx
                    hym = (1.0 - ly) * msk
                    lym = ly * msk
                    x1 = x0 + 1.0
                    y1 = y0 + 1.0
                    vx0 = (x0 >= 0.0) & (x0 < float(_W))
                    vx1 = (x1 >= 0.0) & (x1 < float(_W))
                    vy0 = (y0 >= 0.0) & (y0 < float(_H))
                    vy1 = (y1 >= 0.0) & (y1 < float(_H))
                    xc0 = jnp.clip(x0, 0.0, float(_W - 1))
                    xc1 = jnp.clip(x1, 0.0, float(_W - 1))
                    ly0 = jnp.clip(y0, 0.0, float(_H - 1)) * float(_W)
                    ly1 = jnp.clip(y1, 0.0, float(_H - 1)) * float(_W)
                    for liny, vy, xc, vx, bw in (
                        (ly0, vy0, xc0, vx0, hym * hx),
                        (ly0, vy0, xc1, vx1, hym * lx),
                        (ly1, vy1, xc0, vx0, lym * hx),
                        (ly1, vy1, xc1, vx1, lym * lx),
                    ):
                        lin = (liny + xc).astype(jnp.int32)
                        wv = jnp.where(vy & vx, bw, 0.0)
                        for ch in range(_GC):
                            cvec = jnp.full((16,), ch, jnp.int32)
                            val = plsc.load_gather(xvt, [cvec, lin])
                            acc[ch] = acc[ch] + wv * val
                for ch in range(_GC):
                    outv[ch, pl.ds(l0, 16)] = acc[ch]
            return c2

        lax.fori_loop(0, _L // 16, bbody, 0)
        g = j * 3 + t
        pending_out[slot] = pltpu.async_copy(
            outv, out_hbm.at[n, pl.ds(g * _GC, _GC), :], so[slot]
        )
    for h in pending_out:
        if h is not None:
            h.wait()


def kernel(input, value_w, value_b, offset_mask_w, offset_mask_b, output_w, output_b):
    x_t = jnp.transpose(input, (0, 2, 1))
    w2 = offset_mask_w[_perm] * _keep
    b2 = offset_mask_b[_perm] * _keep[:, 0] + _bshift
    w_all = jnp.concatenate([value_w, w2], axis=0)
    b_all = jnp.concatenate([value_b, b2], axis=0)[:, None]
    a = _proj(w_all, x_t, b_all, _ROWS_A)
    s = _sc_sample(a)
    c = _proj(output_w, s, output_b[:, None], _CH)
    return jnp.transpose(c, (0, 2, 1))
